# Initial kernel scaffold; baseline (speedup 1.0000x reference)
import jax
import jax.numpy as jnp
from jax.experimental import pallas as pl


def kernel(loc_pred, cls_pred, gt_boxes, gt_labels, default_boxes):
    raise NotImplementedError("write your pallas kernel here")


# speedup vs baseline: 29.0426x; 29.0426x over previous
import functools

import jax
import jax.numpy as jnp
from jax.experimental import pallas as pl
from jax.experimental.pallas import tpu as pltpu

B = 64
D = 8732
C = 21
NOBJ = 8
THRESHOLD = 0.5
NEG_POS = 3
ALPHA = 1.0


def _phase_a_kernel(cls_ref, loc_ref, gtb_ref, gtl_ref, db_ref,
                    neg_ref, stats_ref):
    cls = cls_ref[0]
    locp = loc_ref[0]
    gtb = gtb_ref[0]
    gtl = gtl_ref[0]

    cxd = db_ref[0:1, :]
    cyd = db_ref[1:2, :]
    wd = db_ref[2:3, :]
    hd = db_ref[3:4, :]
    x1d = cxd - wd * 0.5
    y1d = cyd - hd * 0.5
    x2d = cxd + wd * 0.5
    y2d = cyd + hd * 0.5
    area_d = (x2d - x1d) * (y2d - y1d)

    x1g = gtb[:, 0:1]
    y1g = gtb[:, 1:2]
    x2g = gtb[:, 2:3]
    y2g = gtb[:, 3:4]
    area_g = (x2g - x1g) * (y2g - y1g)

    iw = jnp.maximum(jnp.minimum(x2g, x2d) - jnp.maximum(x1g, x1d), 0.0)
    ih = jnp.maximum(jnp.minimum(y2g, y2d) - jnp.maximum(y1g, y1d), 0.0)
    inter = iw * ih
    ov = inter / (area_g + area_d - inter)

    obj_iota = jax.lax.broadcasted_iota(jnp.int32, (NOBJ, 1), 0)
    lane_iota = jax.lax.broadcasted_iota(jnp.int32, (1, D), 1)

    bto = jnp.max(ov, axis=0, keepdims=True)
    bti = jnp.min(jnp.where(ov == bto, obj_iota, NOBJ), axis=0,
                  keepdims=True)

    row_max = jnp.max(ov, axis=1, keepdims=True)
    bdi = jnp.min(jnp.where(ov == row_max, lane_iota, D), axis=1,
                  keepdims=True)

    hit = lane_iota == bdi
    override = jnp.max(jnp.where(hit, obj_iota, -1), axis=0,
                       keepdims=True)
    forced = override >= 0
    bti = jnp.where(forced, override, bti)
    bto = jnp.where(forced, 1.0, bto)

    sel = bti == obj_iota
    gtl_col = jnp.transpose(gtl)
    labels = jnp.sum(jnp.where(sel, gtl_col, 0), axis=0, keepdims=True)
    conf = jnp.where(bto < THRESHOLD, 0, labels)
    pos = conf > 0

    gx1 = jnp.sum(jnp.where(sel, x1g, 0.0), axis=0, keepdims=True)
    gy1 = jnp.sum(jnp.where(sel, y1g, 0.0), axis=0, keepdims=True)
    gx2 = jnp.sum(jnp.where(sel, x2g, 0.0), axis=0, keepdims=True)
    gy2 = jnp.sum(jnp.where(sel, y2g, 0.0), axis=0, keepdims=True)

    gcx = (gx1 + gx2) * 0.5
    gcy = (gy1 + gy2) * 0.5
    gw = gx2 - gx1
    gh = gy2 - gy1
    t0 = (gcx - cxd) / (wd / 10.0)
    t1 = (gcy - cyd) / (hd / 10.0)
    t2 = jnp.log(gw / wd) * 5.0
    t3 = jnp.log(gh / hd) * 5.0

    posf = pos.astype(jnp.float32)

    def sl1(p, t):
        d = p - t
        ad = jnp.abs(d)
        return jnp.where(ad < 1.0, 0.5 * ad * ad, ad - 0.5)

    sl1_sum = jnp.sum((sl1(locp[0:1, :], t0) + sl1(locp[1:2, :], t1) +
                       sl1(locp[2:3, :], t2) + sl1(locp[3:4, :], t3)) * posf)

    m = jnp.max(cls, axis=0, keepdims=True)
    s = jnp.sum(jnp.exp(cls - m), axis=0, keepdims=True)
    lse = m + jnp.log(s)
    cls_iota = jax.lax.broadcasted_iota(jnp.int32, (C, 1), 0)
    gathered = jnp.sum(jnp.where(conf == cls_iota, cls, 0.0), axis=0,
                       keepdims=True)
    ce = lse - gathered

    n_pos = jnp.sum(posf)
    pos_loss = jnp.sum(jnp.where(pos, ce, 0.0))
    neg_ref[0, :, :] = jnp.where(pos, 0.0, ce)

    stat_iota = jax.lax.broadcasted_iota(jnp.int32, (1, 128), 1)
    stats_row = (jnp.where(stat_iota == 0, n_pos, 0.0)
                 + jnp.where(stat_iota == 1, pos_loss, 0.0)
                 + jnp.where(stat_iota == 2, sl1_sum, 0.0))
    stats_ref[0, :, :] = stats_row


def _phase_b_kernel(neg_ref, stats_ref, out_ref):
    npos = stats_ref[:, 0:1]
    k_f = NEG_POS * npos
    k_i = k_f.astype(jnp.int32)

    nb = jax.lax.bitcast_convert_type(neg_ref[:, :], jnp.int32)

    def body(i, t_bits):
        shift = 30 - i
        cand = t_bits | jax.lax.shift_left(jnp.int32(1), shift)
        cnt = jnp.sum((nb >= cand).astype(jnp.int32), axis=1, keepdims=True)
        return jnp.where(cnt >= k_i, cand, t_bits)

    t_bits = jax.lax.fori_loop(0, 31, body,
                               jnp.zeros((B, 1), jnp.int32))
    t = jax.lax.bitcast_convert_type(t_bits, jnp.float32)

    neg = neg_ref[:, :]
    gt_mask = neg > t
    cnt_gt = jnp.sum(gt_mask.astype(jnp.float32), axis=1, keepdims=True)
    sum_gt = jnp.sum(jnp.where(gt_mask, neg, 0.0), axis=1, keepdims=True)
    hard = jnp.where(k_i > 0, sum_gt + (k_f - cnt_gt) * t, 0.0)

    n_pos_total = jnp.sum(npos)
    pos_loss = jnp.sum(stats_ref[:, 1:2])
    sl1_sum = jnp.sum(stats_ref[:, 2:3])
    loss = (ALPHA * sl1_sum / (n_pos_total * 4.0)
            + (jnp.sum(hard) + pos_loss) / n_pos_total)
    out_ref[:, :] = loss * jnp.ones((1, 128), jnp.float32)


@functools.partial(jax.jit, static_argnames=("interpret",))
def kernel(loc_pred, cls_pred, gt_boxes, gt_labels, default_boxes,
           interpret=False):
    cls_t = jnp.transpose(cls_pred, (0, 2, 1))
    loc_t = jnp.transpose(loc_pred, (0, 2, 1))
    db_t = jnp.transpose(default_boxes)
    gtl3 = gt_labels.reshape(B, 1, NOBJ)

    neg_ce, stats = pl.pallas_call(
        _phase_a_kernel,
        grid=(B,),
        in_specs=[
            pl.BlockSpec((1, C, D), lambda b: (b, 0, 0)),
            pl.BlockSpec((1, 4, D), lambda b: (b, 0, 0)),
            pl.BlockSpec((1, NOBJ, 4), lambda b: (b, 0, 0)),
            pl.BlockSpec((1, 1, NOBJ), lambda b: (b, 0, 0)),
            pl.BlockSpec((4, D), lambda b: (0, 0)),
        ],
        out_specs=[
            pl.BlockSpec((1, 1, D), lambda b: (b, 0, 0)),
            pl.BlockSpec((1, 1, 128), lambda b: (b, 0, 0)),
        ],
        out_shape=[
            jax.ShapeDtypeStruct((B, 1, D), jnp.float32),
            jax.ShapeDtypeStruct((B, 1, 128), jnp.float32),
        ],
        interpret=interpret,
    )(cls_t, loc_t, gt_boxes, gtl3, db_t)

    out = pl.pallas_call(
        _phase_b_kernel,
        grid=(1,),
        in_specs=[
            pl.BlockSpec((B, D), lambda i: (0, 0)),
            pl.BlockSpec((B, 128), lambda i: (0, 0)),
        ],
        out_specs=pl.BlockSpec((1, 128), lambda i: (0, 0)),
        out_shape=jax.ShapeDtypeStruct((1, 128), jnp.float32),
        interpret=interpret,
    )(neg_ce.reshape(B, D), stats.reshape(B, 128))

    return out[0, 0]


# speedup vs baseline: 38.4967x; 1.3255x over previous
import functools

import jax
import jax.numpy as jnp
from jax.experimental import pallas as pl
from jax.experimental.pallas import tpu as pltpu

B = 64
D = 8732
C = 21
NOBJ = 8
THRESHOLD = 0.5
NEG_POS = 3
ALPHA = 1.0


def _phase_a_kernel(cls_ref, loc_ref, gtb_ref, gtbt_ref, gtl_ref, db_ref,
                    neg_ref, stats_ref):
    cls = cls_ref[0]
    locp = loc_ref[0]
    gtb = gtb_ref[0]
    gtbt = gtbt_ref[0]
    gtl = gtl_ref[0]

    cxd = db_ref[0:1, :]
    cyd = db_ref[1:2, :]
    wd = db_ref[2:3, :]
    hd = db_ref[3:4, :]
    x1d = cxd - wd * 0.5
    y1d = cyd - hd * 0.5
    x2d = cxd + wd * 0.5
    y2d = cyd + hd * 0.5
    area_d = (x2d - x1d) * (y2d - y1d)

    x1g = gtb[:, 0:1]
    y1g = gtb[:, 1:2]
    x2g = gtb[:, 2:3]
    y2g = gtb[:, 3:4]
    area_g = (x2g - x1g) * (y2g - y1g)

    iw = jnp.maximum(jnp.minimum(x2g, x2d) - jnp.maximum(x1g, x1d), 0.0)
    ih = jnp.maximum(jnp.minimum(y2g, y2d) - jnp.maximum(y1g, y1d), 0.0)
    inter = iw * ih
    ov = inter / (area_g + area_d - inter)

    obj_iota = jax.lax.broadcasted_iota(jnp.int32, (NOBJ, 1), 0)
    lane_iota = jax.lax.broadcasted_iota(jnp.int32, (1, D), 1)

    bto = jnp.max(ov, axis=0, keepdims=True)
    bti = jnp.min(jnp.where(ov == bto, obj_iota, NOBJ), axis=0,
                  keepdims=True)

    row_max = jnp.max(ov, axis=1, keepdims=True)
    bdi = jnp.min(jnp.where(ov == row_max, lane_iota, D), axis=1,
                  keepdims=True)

    hit = lane_iota == bdi
    override = jnp.max(jnp.where(hit, obj_iota, -1), axis=0,
                       keepdims=True)
    forced = override >= 0
    bti = jnp.where(forced, override, bti)
    bto = jnp.where(forced, 1.0, bto)

    sel = (bti == obj_iota).astype(jnp.float32)
    labels = jax.lax.dot_general(
        gtl, sel, (((1,), (0,)), ((), ())),
        preferred_element_type=jnp.float32)
    g = jax.lax.dot_general(
        gtbt, sel, (((1,), (0,)), ((), ())),
        preferred_element_type=jnp.float32)

    conf = jnp.where(bto < THRESHOLD, 0, labels.astype(jnp.int32))
    pos = conf > 0

    ri = jax.lax.broadcasted_iota(jnp.int32, (4, 4), 0)
    ci = jax.lax.broadcasted_iota(jnp.int32, (4, 4), 1)
    amat = jnp.where((ri & 1) == (ci & 1),
                     jnp.where(ri < 2, 0.5,
                               jnp.where(ci >= 2, 1.0, -1.0)),
                     0.0).astype(jnp.float32)
    u = jax.lax.dot_general(
        amat, g, (((1,), (0,)), ((), ())),
        preferred_element_type=jnp.float32)

    dbwh = jnp.concatenate([db_ref[2:4, :], db_ref[2:4, :]], axis=0)
    row_iota = jax.lax.broadcasted_iota(jnp.int32, (4, 1), 0)
    tmat = jnp.where(row_iota < 2,
                     (u - db_ref[:, :]) / (dbwh / 10.0),
                     jnp.log(u / dbwh) * 5.0)

    posf = pos.astype(jnp.float32)

    d4 = locp - tmat
    ad4 = jnp.abs(d4)
    sl1v = jnp.where(ad4 < 1.0, 0.5 * ad4 * ad4, ad4 - 0.5)
    sl1_sum = jnp.sum(sl1v * posf)

    m = jnp.max(cls, axis=0, keepdims=True)
    e = jnp.exp(cls - m)
    ones_row = jnp.ones((1, C), jnp.float32)
    s = jax.lax.dot_general(
        ones_row, e, (((1,), (0,)), ((), ())),
        preferred_element_type=jnp.float32)
    lse = m + jnp.log(s)
    cls_iota = jax.lax.broadcasted_iota(jnp.int32, (C, 1), 0)
    gathered = jnp.sum(jnp.where(conf == cls_iota, cls, 0.0), axis=0,
                       keepdims=True)
    ce = lse - gathered

    n_pos = jnp.sum(posf)
    pos_loss = jnp.sum(jnp.where(pos, ce, 0.0))
    neg_ref[0, :, :] = jnp.where(pos, 0.0, ce)

    stat_iota = jax.lax.broadcasted_iota(jnp.int32, (1, 128), 1)
    stats_row = (jnp.where(stat_iota == 0, n_pos, 0.0)
                 + jnp.where(stat_iota == 1, pos_loss, 0.0)
                 + jnp.where(stat_iota == 2, sl1_sum, 0.0))
    stats_ref[0, :, :] = stats_row


def _phase_b_kernel(neg_ref, stats_ref, out_ref):
    npos = stats_ref[:, 0:1]
    k_f = NEG_POS * npos
    k_i = k_f.astype(jnp.int32)

    nb = jax.lax.bitcast_convert_type(neg_ref[:, :], jnp.int32)

    def body(i, t_bits):
        shift = 30 - i
        cand = t_bits | jax.lax.shift_left(jnp.int32(1), shift)
        cnt = jnp.sum((nb >= cand).astype(jnp.int32), axis=1, keepdims=True)
        return jnp.where(cnt >= k_i, cand, t_bits)

    t_bits = jax.lax.fori_loop(0, 31, body,
                               jnp.zeros((B, 1), jnp.int32))
    t = jax.lax.bitcast_convert_type(t_bits, jnp.float32)

    neg = neg_ref[:, :]
    gt_mask = neg > t
    cnt_gt = jnp.sum(gt_mask.astype(jnp.float32), axis=1, keepdims=True)
    sum_gt = jnp.sum(jnp.where(gt_mask, neg, 0.0), axis=1, keepdims=True)
    hard = jnp.where(k_i > 0, sum_gt + (k_f - cnt_gt) * t, 0.0)

    n_pos_total = jnp.sum(npos)
    pos_loss = jnp.sum(stats_ref[:, 1:2])
    sl1_sum = jnp.sum(stats_ref[:, 2:3])
    loss = (ALPHA * sl1_sum / (n_pos_total * 4.0)
            + (jnp.sum(hard) + pos_loss) / n_pos_total)
    out_ref[:, :] = loss * jnp.ones((1, 128), jnp.float32)


@functools.partial(jax.jit, static_argnames=("interpret",))
def kernel(loc_pred, cls_pred, gt_boxes, gt_labels, default_boxes,
           interpret=False):
    cls_t = jnp.transpose(cls_pred, (0, 2, 1))
    loc_t = jnp.transpose(loc_pred, (0, 2, 1))
    db_t = jnp.transpose(default_boxes)
    gtb_t = jnp.transpose(gt_boxes, (0, 2, 1))
    gtl3 = gt_labels.astype(jnp.float32).reshape(B, 1, NOBJ)

    neg_ce, stats = pl.pallas_call(
        _phase_a_kernel,
        grid=(B,),
        in_specs=[
            pl.BlockSpec((1, C, D), lambda b: (b, 0, 0)),
            pl.BlockSpec((1, 4, D), lambda b: (b, 0, 0)),
            pl.BlockSpec((1, NOBJ, 4), lambda b: (b, 0, 0)),
            pl.BlockSpec((1, 4, NOBJ), lambda b: (b, 0, 0)),
            pl.BlockSpec((1, 1, NOBJ), lambda b: (b, 0, 0)),
            pl.BlockSpec((4, D), lambda b: (0, 0)),
        ],
        out_specs=[
            pl.BlockSpec((1, 1, D), lambda b: (b, 0, 0)),
            pl.BlockSpec((1, 1, 128), lambda b: (b, 0, 0)),
        ],
        out_shape=[
            jax.ShapeDtypeStruct((B, 1, D), jnp.float32),
            jax.ShapeDtypeStruct((B, 1, 128), jnp.float32),
        ],
        interpret=interpret,
    )(cls_t, loc_t, gt_boxes, gtb_t, gtl3, db_t)

    out = pl.pallas_call(
        _phase_b_kernel,
        grid=(1,),
        in_specs=[
            pl.BlockSpec((B, D), lambda i: (0, 0)),
            pl.BlockSpec((B, 128), lambda i: (0, 0)),
        ],
        out_specs=pl.BlockSpec((1, 128), lambda i: (0, 0)),
        out_shape=jax.ShapeDtypeStruct((1, 128), jnp.float32),
        interpret=interpret,
    )(neg_ce.reshape(B, D), stats.reshape(B, 128))

    return out[0, 0]
